# baseline (speedup 1.0000x reference)
import functools

import jax
import jax.numpy as jnp
from jax import lax
from jax.experimental import pallas as pl
from jax.experimental.pallas import tpu as pltpu
from jax.experimental.pallas import tpu_sc as plsc

N = 50000
E = 800000
C = 128
H = 64
HH = 32

NT = 16
CH = 128
KB = 8
CHG = 128
NG = KB * CH // CHG
RPC = CHG // CH
NS = 6
NCH_T = 392
E_PAD = NT * NCH_T * CH
NB = NCH_T // KB
N_PAD = 50048
RPT = N_PAD // NT

BN_A = 1000
NBLK_A = N // BN_A
BN_B = RPT
NBLK_B = NT
HB = NT


def _support_body(x_ref, wc_ref, wp_ref, oc_ref, op_ref):
    xb = x_ref[...]
    sc = jnp.dot(xb, wc_ref[...], preferred_element_type=jnp.float32)
    sp = jnp.dot(xb, wp_ref[...], preferred_element_type=jnp.float32)
    oc_ref[0] = sc[:, :HH]
    oc_ref[1] = sc[:, HH:]
    op_ref[0] = sp[:, :HH]
    op_ref[1] = sp[:, HH:]


def _supports(x, Wc, Wp):
    oc, op = pl.pallas_call(
        _support_body,
        grid=(NBLK_A,),
        in_specs=[
            pl.BlockSpec((BN_A, C), lambda i: (i, 0)),
            pl.BlockSpec((C, H), lambda i: (0, 0)),
            pl.BlockSpec((C, H), lambda i: (0, 0)),
        ],
        out_specs=[
            pl.BlockSpec((2, BN_A, HH), lambda i: (0, i, 0)),
            pl.BlockSpec((2, BN_A, HH), lambda i: (0, i, 0)),
        ],
        out_shape=[
            jax.ShapeDtypeStruct((2, N, HH), jnp.float32),
            jax.ShapeDtypeStruct((2, N, HH), jnp.float32),
        ],
    )(x, Wc, Wp)
    return oc.reshape(2 * N, HH), op.reshape(2 * N, HH)


def _spmm_body(zrs, sup_c, sup_p, combo_c, combo_p,
               out_c, out_p, acc, cidx, rows, gsem, ssem, isem):
    cid = lax.axis_index("c")
    sid = lax.axis_index("s")
    cofs = cid * N
    oofs = cid * N_PAD

    def run_layer(sup, combom, out):
        pltpu.sync_copy(zrs, acc.at[pl.ds(sid * RPT, RPT)])
        plsc.subcore_barrier()

        def fire_idx(b, p):
            pltpu.async_copy(combom.at[sid * NB + b], cidx.at[p], isem)

        def wait_idx(b, p):
            pltpu.make_async_copy(
                combom.at[sid * NB + b], cidx.at[p], isem).wait()

        fire_idx(jnp.int32(0), jnp.int32(0))

        def batch_body(b, carry):
            p = lax.rem(b, 2)
            wait_idx(b, p)
            for j in range(KB):
                for t in range(CH // 16):
                    cidx[p, 0, j, pl.ds(t * 16, 16)] = (
                        cidx[p, 0, j, pl.ds(t * 16, 16)] + cofs)
            def fire_g(k):
                return pltpu.async_copy(
                    sup.at[cidx.at[p, 0, k]],
                    rows.at[k % NS], gsem.at[k % NS])
            gd = {k: fire_g(k) for k in range(NS)}
            fire_idx(b + 1, 1 - p)
            sd = {}
            for k in range(NG):
                s = k % NS
                gd[k].wait()

                def scale(g, c3, k=k, s=s):
                    vals16 = lax.bitcast_convert_type(
                        cidx[p, 2, k, pl.ds(g * 16, 16)], jnp.float32)
                    for t in range(16):
                        e = g * 16 + t
                        bv = jnp.full((16,), vals16[t])
                        rows[s, e, pl.ds(0, 16)] = (
                            rows[s, e, pl.ds(0, 16)] * bv)
                        rows[s, e, pl.ds(16, 16)] = (
                            rows[s, e, pl.ds(16, 16)] * bv)
                    return c3
                lax.fori_loop(0, CHG // 16, scale, 0)
                sd[k] = [pltpu.async_copy(
                    rows.at[s, pl.ds(h * CH, CH)],
                    acc.at[cidx.at[p, 1, k * RPC + h]],
                    ssem.at[s * RPC + h], add=True) for h in range(RPC)]
                if 1 <= k:
                    for d in sd[k - 1]:
                        d.wait()
                    if k + NS - 1 < NG:
                        gd[k + NS - 1] = fire_g(k + NS - 1)
            for d in sd[NG - 1]:
                d.wait()
            return carry
        lax.fori_loop(0, NB, batch_body, 0)
        wait_idx(jnp.int32(NB), jnp.int32(NB % 2))
        plsc.subcore_barrier()
        pltpu.sync_copy(acc.at[pl.ds(sid * RPT, RPT)],
                        out.at[pl.ds(oofs + sid * RPT, RPT)])

    run_layer(sup_c, combo_c, out_c)
    run_layer(sup_p, combo_p, out_p)


def _spmm(sup_c2, sup_p2, ec, vc, ep, vp):
    def prep(edge, vals):
        pad = E_PAD + KB * CH - E
        src = jnp.concatenate([edge[0], jnp.zeros((pad,), jnp.int32)])
        dst = jnp.concatenate([edge[1], jnp.zeros((pad,), jnp.int32)])
        val = lax.bitcast_convert_type(
            jnp.concatenate([vals, jnp.zeros((pad,), jnp.float32)]),
            jnp.int32)
        return jnp.stack([src.reshape(-1, KB, CH), dst.reshape(-1, KB, CH),
                          val.reshape(-1, KB, CH)], axis=1)

    combo_c = prep(ec, vc)
    combo_p = prep(ep, vp)
    zrs = jnp.zeros((RPT, HH), jnp.float32)

    fn = pl.kernel(
        _spmm_body,
        out_type=[
            jax.ShapeDtypeStruct((2 * N_PAD, HH), jnp.float32),
            jax.ShapeDtypeStruct((2 * N_PAD, HH), jnp.float32),
        ],
        mesh=plsc.VectorSubcoreMesh(core_axis_name="c", subcore_axis_name="s"),
        compiler_params=pltpu.CompilerParams(use_tc_tiling_on_sc=False),
        scratch_types=[
            pltpu.VMEM_SHARED((N_PAD, HH), jnp.float32),
            pltpu.VMEM((2, 3, KB, CH), jnp.int32),
            pltpu.VMEM((NS, CHG, HH), jnp.float32),
            pltpu.SemaphoreType.DMA((NS,)),
            pltpu.SemaphoreType.DMA((2 * NS,)),
            pltpu.SemaphoreType.DMA,
        ],
    )
    return fn(zrs, sup_c2, sup_p2, combo_c, combo_p)


def _att_body(cl_ref, ch_ref, pl_ref, ph_ref, bc_ref, bp_ref, wct_ref,
              wpt_ref, wout_ref, bout_ref, o_ref):
    xc = jnp.concatenate([cl_ref[...], ch_ref[...]], axis=1) + bc_ref[...]
    xp = jnp.concatenate([pl_ref[...], ph_ref[...]], axis=1) + bp_ref[...]
    cat = jnp.concatenate([xc, xp], axis=1)
    s_c = jnp.sum(cat * wct_ref[...], axis=1, keepdims=True)
    s_p = jnp.sum(cat * wpt_ref[...], axis=1, keepdims=True)
    m = jnp.maximum(s_c, s_p)
    e_c = jnp.exp(s_c - m)
    e_p = jnp.exp(s_p - m)
    h = jnp.maximum((e_c * xc + e_p * xp) / (e_c + e_p), 0.0)
    o_ref[...] = lax.dot_general(
        h, wout_ref[...], (((1,), (1,)), ((), ())),
        preferred_element_type=jnp.float32) + bout_ref[...]


def _attention(acc_c2, acc_p2, bc, bp, wct, wpt, W_out, b_out):
    full = lambda shape: pl.BlockSpec(shape, lambda i: (0, 0))
    return pl.pallas_call(
        _att_body,
        grid=(NBLK_B,),
        in_specs=[
            pl.BlockSpec((BN_B, HH), lambda i: (i, 0)),
            pl.BlockSpec((BN_B, HH), lambda i: (i + HB, 0)),
            pl.BlockSpec((BN_B, HH), lambda i: (i, 0)),
            pl.BlockSpec((BN_B, HH), lambda i: (i + HB, 0)),
            full((1, H)), full((1, H)), full((1, 2 * H)), full((1, 2 * H)),
            full((C, H)), full((1, C)),
        ],
        out_specs=pl.BlockSpec((BN_B, C), lambda i: (i, 0)),
        out_shape=jax.ShapeDtypeStruct((N, C), jnp.float32),
    )(acc_c2, acc_c2, acc_p2, acc_p2, bc.reshape(1, H), bp.reshape(1, H),
      wct, wpt, W_out, b_out.reshape(1, C))


def kernel(x, Wc, bc, Wp, bp, w_chem_t, w_protein_t, W_out, b_out,
           vals_chem, vals_protein, edge_chem, edge_protein):
    sup_c2, sup_p2 = _supports(x, Wc, Wp)
    acc_c2, acc_p2 = _spmm(sup_c2, sup_p2, edge_chem, vals_chem,
                           edge_protein, vals_protein)
    return _attention(acc_c2, acc_p2, bc, bp, w_chem_t, w_protein_t,
                      W_out, b_out)


# speedup vs baseline: 1.0086x; 1.0086x over previous
import functools

import jax
import jax.numpy as jnp
from jax import lax
from jax.experimental import pallas as pl
from jax.experimental.pallas import tpu as pltpu
from jax.experimental.pallas import tpu_sc as plsc

N = 50000
E = 800000
C = 128
H = 64
HH = 32

NT = 16
CH = 128
KB = 8
CHG = 128
NG = KB * CH // CHG
RPC = CHG // CH
NS = 6
NCH_T = 392
E_PAD = NT * NCH_T * CH
NB = NCH_T // KB
N_PAD = 50048
RPT = N_PAD // NT

BN_A = 1000
NBLK_A = N // BN_A
BN_B = RPT
NBLK_B = NT
HB = NT


def _support_body(x_ref, wc_ref, wp_ref, oc_ref, op_ref):
    xb = x_ref[...]
    sc = jnp.dot(xb, wc_ref[...], preferred_element_type=jnp.float32)
    sp = jnp.dot(xb, wp_ref[...], preferred_element_type=jnp.float32)
    oc_ref[0] = sc[:, :HH]
    oc_ref[1] = sc[:, HH:]
    op_ref[0] = sp[:, :HH]
    op_ref[1] = sp[:, HH:]


def _supports(x, Wc, Wp):
    oc, op = pl.pallas_call(
        _support_body,
        grid=(NBLK_A,),
        in_specs=[
            pl.BlockSpec((BN_A, C), lambda i: (i, 0)),
            pl.BlockSpec((C, H), lambda i: (0, 0)),
            pl.BlockSpec((C, H), lambda i: (0, 0)),
        ],
        out_specs=[
            pl.BlockSpec((2, BN_A, HH), lambda i: (0, i, 0)),
            pl.BlockSpec((2, BN_A, HH), lambda i: (0, i, 0)),
        ],
        out_shape=[
            jax.ShapeDtypeStruct((2, N, HH), jnp.float32),
            jax.ShapeDtypeStruct((2, N, HH), jnp.float32),
        ],
    )(x, Wc, Wp)
    return oc.reshape(2 * N, HH), op.reshape(2 * N, HH)


def _spmm_body(zrs, sup_c, sup_p, src_c, dst_c, vals_c, src_p, dst_p, vals_p,
               out_c, out_p, acc, src_v, dst_v, vals_v, rows,
               gsem, ssem, isem):
    cid = lax.axis_index("c")
    sid = lax.axis_index("s")
    cofs = cid * N
    oofs = cid * N_PAD

    def run_layer(sup, srcm, dstm, valsm, out):
        pltpu.sync_copy(zrs, acc.at[pl.ds(sid * RPT, RPT)])
        plsc.subcore_barrier()

        def fire_idx(b, p):
            row0 = sid * NCH_T + b * KB
            pltpu.async_copy(
                srcm.at[pl.ds(row0 * CH, KB * CH)], src_v.at[p], isem)
            pltpu.async_copy(dstm.at[pl.ds(row0, KB)], dst_v.at[p], isem)
            pltpu.async_copy(
                valsm.at[pl.ds(row0 * CH, KB * CH)], vals_v.at[p], isem)

        def wait_idx(b, p):
            row0 = sid * NCH_T + b * KB
            pltpu.make_async_copy(
                srcm.at[pl.ds(row0 * CH, KB * CH)], src_v.at[p], isem).wait()
            pltpu.make_async_copy(
                dstm.at[pl.ds(row0, KB)], dst_v.at[p], isem).wait()
            pltpu.make_async_copy(
                valsm.at[pl.ds(row0 * CH, KB * CH)], vals_v.at[p],
                isem).wait()

        fire_idx(jnp.int32(0), jnp.int32(0))

        def batch_body(b, carry):
            p = lax.rem(b, 2)
            wait_idx(b, p)
            for u in range(KB * CH // 16):
                src_v[p, pl.ds(u * 16, 16)] = (
                    src_v[p, pl.ds(u * 16, 16)] + cofs)
            def fire_g(k):
                return pltpu.async_copy(
                    sup.at[src_v.at[p, pl.ds(k * CHG, CHG)]],
                    rows.at[k % NS], gsem.at[k % NS])
            gd = {k: fire_g(k) for k in range(NS)}
            fire_idx(b + 1, 1 - p)
            sd = {}
            for k in range(NG):
                s = k % NS
                gd[k].wait()

                def scale(g, c3, k=k, s=s):
                    vals16 = vals_v[p, pl.ds(k * CHG + g * 16, 16)]
                    for t in range(16):
                        e = g * 16 + t
                        bv = jnp.full((16,), vals16[t])
                        rows[s, e, pl.ds(0, 16)] = (
                            rows[s, e, pl.ds(0, 16)] * bv)
                        rows[s, e, pl.ds(16, 16)] = (
                            rows[s, e, pl.ds(16, 16)] * bv)
                    return c3
                lax.fori_loop(0, CHG // 16, scale, 0)
                sd[k] = [pltpu.async_copy(
                    rows.at[s, pl.ds(h * CH, CH)],
                    acc.at[dst_v.at[p, k * RPC + h]],
                    ssem.at[s * RPC + h], add=True) for h in range(RPC)]
                if 1 <= k:
                    for d in sd[k - 1]:
                        d.wait()
                    if k + NS - 1 < NG:
                        gd[k + NS - 1] = fire_g(k + NS - 1)
            for d in sd[NG - 1]:
                d.wait()
            return carry
        lax.fori_loop(0, NB, batch_body, 0)
        wait_idx(jnp.int32(NB), jnp.int32(NB % 2))
        plsc.subcore_barrier()
        pltpu.sync_copy(acc.at[pl.ds(sid * RPT, RPT)],
                        out.at[pl.ds(oofs + sid * RPT, RPT)])

    run_layer(sup_c, src_c, dst_c, vals_c, out_c)
    run_layer(sup_p, src_p, dst_p, vals_p, out_p)


def _spmm(sup_c2, sup_p2, ec, vc, ep, vp):
    def prep(edge, vals):
        pad = E_PAD + KB * CH - E
        src = jnp.concatenate([edge[0], jnp.zeros((pad,), jnp.int32)])
        dst = jnp.concatenate([edge[1], jnp.zeros((pad,), jnp.int32)])
        val = jnp.concatenate([vals, jnp.zeros((pad,), jnp.float32)])
        return src, dst.reshape(-1, CH), val

    sc_, dc_, vc_ = prep(ec, vc)
    sp_, dp_, vp_ = prep(ep, vp)
    zrs = jnp.zeros((RPT, HH), jnp.float32)

    fn = pl.kernel(
        _spmm_body,
        out_type=[
            jax.ShapeDtypeStruct((2 * N_PAD, HH), jnp.float32),
            jax.ShapeDtypeStruct((2 * N_PAD, HH), jnp.float32),
        ],
        mesh=plsc.VectorSubcoreMesh(core_axis_name="c", subcore_axis_name="s"),
        compiler_params=pltpu.CompilerParams(use_tc_tiling_on_sc=False),
        scratch_types=[
            pltpu.VMEM_SHARED((N_PAD, HH), jnp.float32),
            pltpu.VMEM((2, KB * CH), jnp.int32),
            pltpu.VMEM((2, KB, CH), jnp.int32),
            pltpu.VMEM((2, KB * CH), jnp.float32),
            pltpu.VMEM((NS, CHG, HH), jnp.float32),
            pltpu.SemaphoreType.DMA((NS,)),
            pltpu.SemaphoreType.DMA((2 * NS,)),
            pltpu.SemaphoreType.DMA,
        ],
    )
    return fn(zrs, sup_c2, sup_p2, sc_, dc_, vc_, sp_, dp_, vp_)


def _att_body(cl_ref, ch_ref, pl_ref, ph_ref, bc_ref, bp_ref, wct_ref,
              wpt_ref, wout_ref, bout_ref, o_ref):
    xc = jnp.concatenate([cl_ref[...], ch_ref[...]], axis=1) + bc_ref[...]
    xp = jnp.concatenate([pl_ref[...], ph_ref[...]], axis=1) + bp_ref[...]
    cat = jnp.concatenate([xc, xp], axis=1)
    s_c = jnp.sum(cat * wct_ref[...], axis=1, keepdims=True)
    s_p = jnp.sum(cat * wpt_ref[...], axis=1, keepdims=True)
    m = jnp.maximum(s_c, s_p)
    e_c = jnp.exp(s_c - m)
    e_p = jnp.exp(s_p - m)
    h = jnp.maximum((e_c * xc + e_p * xp) / (e_c + e_p), 0.0)
    o_ref[...] = lax.dot_general(
        h, wout_ref[...], (((1,), (1,)), ((), ())),
        preferred_element_type=jnp.float32) + bout_ref[...]


def _attention(acc_c2, acc_p2, bc, bp, wct, wpt, W_out, b_out):
    full = lambda shape: pl.BlockSpec(shape, lambda i: (0, 0))
    return pl.pallas_call(
        _att_body,
        grid=(NBLK_B,),
        in_specs=[
            pl.BlockSpec((BN_B, HH), lambda i: (i, 0)),
            pl.BlockSpec((BN_B, HH), lambda i: (i + HB, 0)),
            pl.BlockSpec((BN_B, HH), lambda i: (i, 0)),
            pl.BlockSpec((BN_B, HH), lambda i: (i + HB, 0)),
            full((1, H)), full((1, H)), full((1, 2 * H)), full((1, 2 * H)),
            full((C, H)), full((1, C)),
        ],
        out_specs=pl.BlockSpec((BN_B, C), lambda i: (i, 0)),
        out_shape=jax.ShapeDtypeStruct((N, C), jnp.float32),
    )(acc_c2, acc_c2, acc_p2, acc_p2, bc.reshape(1, H), bp.reshape(1, H),
      wct, wpt, W_out, b_out.reshape(1, C))


def kernel(x, Wc, bc, Wp, bp, w_chem_t, w_protein_t, W_out, b_out,
           vals_chem, vals_protein, edge_chem, edge_protein):
    sup_c2, sup_p2 = _supports(x, Wc, Wp)
    acc_c2, acc_p2 = _spmm(sup_c2, sup_p2, edge_chem, vals_chem,
                           edge_protein, vals_protein)
    return _attention(acc_c2, acc_p2, bc, bp, w_chem_t, w_protein_t,
                      W_out, b_out)
